# baseline (speedup 1.0000x reference)
import functools
import math

import jax
import jax.numpy as jnp
from jax import lax
from jax.experimental import pallas as pl
from jax.experimental.pallas import tpu as pltpu
from jax.experimental.pallas import tpu_sc as plsc

_NW = 32


_SUP = 2


def _sc_gather(idx2d, table):
    nrows = idx2d.shape[0]
    d = table.shape[1]
    rows_w = nrows // _NW
    chb = 2 * _SUP
    nouter = rows_w // chb
    assert rows_w * _NW == nrows and nouter * chb == rows_w
    mesh = plsc.VectorSubcoreMesh(core_axis_name="c", subcore_axis_name="s")

    @functools.partial(
        pl.kernel,
        mesh=mesh,
        out_type=jax.ShapeDtypeStruct((nrows, 128, d), table.dtype),
        scratch_types=[
            pltpu.VMEM((128,), jnp.int32),
            pltpu.VMEM((128,), jnp.int32),
            pltpu.VMEM((128,), jnp.int32),
            pltpu.VMEM((128,), jnp.int32),
            pltpu.VMEM((chb, 128, d), table.dtype),
            pltpu.SemaphoreType.DMA,
        ],
    )
    def k(idx_hbm, table_hbm, out_hbm, i0, i1, i2, i3, rows_v, gsem):
        ibufs = (i0, i1, i2, i3)
        w = lax.axis_index("s") * 2 + lax.axis_index("c")

        def body(o, carry):
            c0 = chb * w + (_NW * chb) * o
            for t in range(chb):
                pltpu.sync_copy(idx_hbm.at[c0 + t], ibufs[t])
            hs = [
                pltpu.async_copy(table_hbm.at[ibufs[t]], rows_v.at[t], gsem)
                for t in range(chb)
            ]
            for h in hs:
                h.wait()
            pltpu.sync_copy(rows_v, out_hbm.at[pl.ds(c0, chb)])
            return carry

        lax.fori_loop(0, nouter, body, 0)

    return k(idx2d, table)


def _tc_linear(fflat, Wt, b2, E, tile):
    nb = E // tile
    out_f = Wt.shape[2]
    F = Wt.shape[1]

    def body(f0, f1, f2, f3, f4, wref, bref, oref):
        a0 = f0[...].astype(jnp.bfloat16)
        a1, a2, a3, a4 = f1[...], f2[...], f3[...], f4[...]
        cs = (
            a0,
            (a1 + a3).astype(jnp.bfloat16),
            (a2 + a4).astype(jnp.bfloat16),
            jnp.abs(a1 - a3).astype(jnp.bfloat16),
            jnp.abs(a2 - a4).astype(jnp.bfloat16),
        )
        acc = jnp.broadcast_to(bref[...].astype(jnp.float32), (tile, out_f))
        for i, c in enumerate(cs):
            acc += jax.lax.dot_general(
                c,
                wref[i],
                (((1,), (0,)), ((), ())),
                preferred_element_type=jnp.float32,
            )
        oref[...] = acc

    fspec = [
        pl.BlockSpec((tile, F), lambda i, k=k: (k * nb + i, 0)) for k in range(5)
    ]
    return pl.pallas_call(
        body,
        grid=(nb,),
        in_specs=fspec
        + [
            pl.BlockSpec((5, F, out_f), lambda i: (0, 0, 0)),
            pl.BlockSpec((1, out_f), lambda i: (0, 0)),
        ],
        out_specs=pl.BlockSpec((tile, out_f), lambda i: (i, 0)),
        out_shape=jax.ShapeDtypeStruct((E, out_f), jnp.float32),
    )(fflat, fflat, fflat, fflat, fflat, Wt, b2)


def kernel(x, edgemat, W, b):
    _, F, E, _ = x.shape
    K = edgemat.shape[2]
    out_f = W.shape[0]
    nslice = 1
    tile = 1280
    es = E // nslice
    cpp = es // 128
    xt = jnp.transpose(x[0, :, :, 0])
    idx3 = jnp.transpose(edgemat[0]).reshape(K, E // 128, 128)
    Wt = jnp.transpose(W.reshape(out_f, K, F), (1, 2, 0)).astype(jnp.bfloat16)
    b2 = b.reshape(1, -1)
    align = math.lcm(_NW * 2 * _SUP, tile // 128)
    ys = []
    for s in range(nslice):
        idx_s = idx3[:, s * cpp : (s + 1) * cpp].reshape(K * cpp, 128)
        nrows = idx_s.shape[0]
        npad = (nrows + align - 1) // align * align
        idx_pad = jnp.concatenate(
            [idx_s, jnp.zeros((npad - nrows, 128), jnp.int32)], axis=0
        )
        fg = _sc_gather(idx_pad, xt)
        ys.append(_tc_linear(fg.reshape(npad * 128, F), Wt, b2, es, tile))
    y = jnp.concatenate(ys, axis=0)
    return jnp.transpose(y)[None, :, :, None]


# speedup vs baseline: 1.5629x; 1.5629x over previous
import functools
import math

import jax
import jax.numpy as jnp
from jax import lax
from jax.experimental import pallas as pl
from jax.experimental.pallas import tpu as pltpu
from jax.experimental.pallas import tpu_sc as plsc

_NW = 32


_SUP = 2


def _sc_gather(idx2d, table):
    nrows = idx2d.shape[0]
    d = table.shape[1]
    rows_w = nrows // _NW
    chb = 2 * _SUP
    nouter = rows_w // chb
    assert rows_w * _NW == nrows and nouter * chb == rows_w
    mesh = plsc.VectorSubcoreMesh(core_axis_name="c", subcore_axis_name="s")

    @functools.partial(
        pl.kernel,
        mesh=mesh,
        out_type=jax.ShapeDtypeStruct((nrows, 128, d), table.dtype),
        scratch_types=[
            pltpu.VMEM((128,), jnp.int32),
            pltpu.VMEM((128,), jnp.int32),
            pltpu.VMEM((128,), jnp.int32),
            pltpu.VMEM((128,), jnp.int32),
            pltpu.VMEM((chb, 128, d), table.dtype),
            pltpu.SemaphoreType.DMA,
        ],
    )
    def k(idx_hbm, table_hbm, out_hbm, i0, i1, i2, i3, rows_v, gsem):
        ibufs = (i0, i1, i2, i3)
        w = lax.axis_index("s") * 2 + lax.axis_index("c")

        def body(o, carry):
            c0 = chb * w + (_NW * chb) * o
            for t in range(chb):
                pltpu.sync_copy(idx_hbm.at[c0 + t], ibufs[t])
            hs = [
                pltpu.async_copy(table_hbm.at[ibufs[t]], rows_v.at[t], gsem)
                for t in range(chb)
            ]
            for h in hs:
                h.wait()
            pltpu.sync_copy(rows_v, out_hbm.at[pl.ds(c0, chb)])
            return carry

        lax.fori_loop(0, nouter, body, 0)

    return k(idx2d, table)


def _tc_linear(fflat, Wt, b2, E, tile):
    nb = E // tile
    out_f = Wt.shape[2]
    F = Wt.shape[1]

    def body(f0, f1, f2, f3, f4, wref, bref, oref):
        a0 = f0[...].astype(jnp.bfloat16)
        a1, a2, a3, a4 = f1[...], f2[...], f3[...], f4[...]
        cs = (
            a0,
            (a1 + a3).astype(jnp.bfloat16),
            (a2 + a4).astype(jnp.bfloat16),
            jnp.abs(a1 - a3).astype(jnp.bfloat16),
            jnp.abs(a2 - a4).astype(jnp.bfloat16),
        )
        acc = jnp.broadcast_to(bref[...].astype(jnp.float32), (tile, out_f))
        for i, c in enumerate(cs):
            acc += jax.lax.dot_general(
                c,
                wref[i],
                (((1,), (0,)), ((), ())),
                preferred_element_type=jnp.float32,
            )
        oref[...] = acc

    fspec = [
        pl.BlockSpec((tile, F), lambda i, k=k: (k * nb + i, 0)) for k in range(5)
    ]
    return pl.pallas_call(
        body,
        grid=(nb,),
        in_specs=fspec
        + [
            pl.BlockSpec((5, F, out_f), lambda i: (0, 0, 0)),
            pl.BlockSpec((1, out_f), lambda i: (0, 0)),
        ],
        out_specs=pl.BlockSpec((tile, out_f), lambda i: (i, 0)),
        out_shape=jax.ShapeDtypeStruct((E, out_f), jnp.float32),
    )(fflat, fflat, fflat, fflat, fflat, Wt, b2)


def kernel(x, edgemat, W, b):
    _, F, E, _ = x.shape
    K = edgemat.shape[2]
    out_f = W.shape[0]
    nslice = 1
    tile = 512
    es = E // nslice
    cpp = es // 128
    xt = jnp.transpose(x[0, :, :, 0])
    idx3 = jnp.transpose(edgemat[0]).reshape(K, E // 128, 128)
    Wt = jnp.transpose(W.reshape(out_f, K, F), (1, 2, 0)).astype(jnp.bfloat16)
    b2 = b.reshape(1, -1)
    align = math.lcm(_NW * 2 * _SUP, tile // 128)
    ys = []
    for s in range(nslice):
        idx_s = idx3[:, s * cpp : (s + 1) * cpp].reshape(K * cpp, 128)
        nrows = idx_s.shape[0]
        npad = (nrows + align - 1) // align * align
        idx_pad = jnp.concatenate(
            [idx_s, jnp.zeros((npad - nrows, 128), jnp.int32)], axis=0
        )
        fg = _sc_gather(idx_pad, xt)
        ys.append(_tc_linear(fg.reshape(npad * 128, F), Wt, b2, es, tile))
    y = jnp.concatenate(ys, axis=0)
    return jnp.transpose(y)[None, :, :, None]
